# baseline (speedup 1.0000x reference)
import functools

import jax
import jax.numpy as jnp
import numpy as np
from jax import lax
from jax.experimental import pallas as pl
from jax.experimental.pallas import tpu as pltpu
from jax.experimental.pallas import tpu_sc as plsc

_NC = 2
_NS = 16
_NW = _NC * _NS
_L = 16
_C = 128
_D = 2 * _C
_HW = _C // 2

_HI_MASK = np.uint32(0xFFFF0000)

_GATHER_DNUMS = lax.GatherDimensionNumbers(
    offset_dims=(), collapsed_slice_dims=(0,), start_index_map=(0,))


def _shuffle(v, idx):
    return lax.gather(v, idx[:, None], _GATHER_DNUMS, (1,),
                      mode=lax.GatherScatterMode.PROMISE_IN_BOUNDS)


def _hsum(v, lanes):
    for k in (8, 4, 2, 1):
        v = v + _shuffle(v, lanes ^ k)
    return v


def _rnd(u):
    return u + np.uint32(0x7FFF) + jnp.bitwise_and(
        jnp.right_shift(u, 16), np.uint32(1))


def _pack_rows(a):
    v = lax.bitcast_convert_type(a, jnp.uint32)
    h = jnp.bitwise_or(
        jnp.bitwise_and(_rnd(v[:, _HW:_C]), _HI_MASK),
        jnp.right_shift(_rnd(v[:, 0:_HW]), 16))
    t = jnp.bitwise_or(
        jnp.bitwise_and(_rnd(v[:, _C + _HW:_D]), _HI_MASK),
        jnp.right_shift(_rnd(v[:, _C:_C + _HW]), 16))
    return jnp.concatenate([h, t], axis=1)


def _sc_body(B, n_chunks, x2, wcat, src, dst, et, out,
             ixs, ixd, ixt, rs_a, rd_a, rw_a, rs_b, rd_b, rw_b, ov_a, ov_b,
             semi, semr_a, semr_b, semo_a, semo_b):
    epw = n_chunks * B
    wid = lax.axis_index("s") * _NC + lax.axis_index("c")
    base = wid * epw
    lanes = lax.iota(jnp.int32, _L)

    def idx_descs():
        return (
            pltpu.make_async_copy(src.at[pl.ds(base, epw)], ixs, semi),
            pltpu.make_async_copy(dst.at[pl.ds(base, epw)], ixd, semi),
            pltpu.make_async_copy(et.at[pl.ds(base, epw)], ixt, semi),
        )

    def row_descs(c, rs, rd, rw, sem):
        off = c * B
        return (
            pltpu.make_async_copy(x2.at[ixs.at[pl.ds(off, B)]], rs, sem),
            pltpu.make_async_copy(x2.at[ixd.at[pl.ds(off, B)]], rd, sem),
            pltpu.make_async_copy(wcat.at[ixt.at[pl.ds(off, B)]], rw, sem),
        )

    def out_desc(c, ov, sem):
        return pltpu.make_async_copy(ov, out.at[pl.ds(base + c * B, B)], sem)

    def fire(descs):
        for d in descs:
            d.start()

    def wait(descs):
        for d in descs:
            d.wait()

    def compute(rs, rd, rw, ov):
        def ext(u):
            hi = plsc.bitcast(jnp.bitwise_and(u, _HI_MASK), jnp.float32)
            lo = plsc.bitcast(jnp.left_shift(u, 16), jnp.float32)
            return hi, lo

        def group(g, gcarry):
            gbase = g * _L
            ovec = jnp.zeros((_L,), jnp.float32)
            for el in range(_L):
                e = gbase + el
                acc1 = jnp.zeros((_L,), jnp.float32)
                acc2 = jnp.zeros((_L,), jnp.float32)
                acc3 = jnp.zeros((_L,), jnp.float32)
                acc4 = jnp.zeros((_L,), jnp.float32)
                for j in range(_HW // _L):
                    lo = j * _L
                    hi = _HW + j * _L
                    sa, sb = ext(rs[e, pl.ds(lo, _L)])
                    da, db = ext(rd[e, pl.ds(hi, _L)])
                    acc1 = acc1 + sa * rw[e, pl.ds(_HW + lo, _L)] * da
                    acc2 = acc2 + sb * rw[e, pl.ds(lo, _L)] * db
                    sa, sb = ext(rs[e, pl.ds(hi, _L)])
                    da, db = ext(rd[e, pl.ds(lo, _L)])
                    acc3 = acc3 + da * rw[e, pl.ds(_C + _HW + lo, _L)] * sa
                    acc4 = acc4 + db * rw[e, pl.ds(_C + lo, _L)] * sb
                acc = (acc1 + acc2) + (acc3 + acc4)
                ovec = jnp.where(lanes == el, _hsum(acc, lanes), ovec)
            ov[pl.ds(gbase, _L)] = ovec
            return gcarry

        lax.fori_loop(0, B // _L, group, 0, unroll=False)

    fire(idx_descs())
    wait(idx_descs())
    fire(row_descs(0, rs_a, rd_a, rw_a, semr_a))

    def step(k, carry):
        c0 = 2 * k
        fire(row_descs(c0 + 1, rs_b, rd_b, rw_b, semr_b))
        wait(row_descs(c0, rs_a, rd_a, rw_a, semr_a))

        @pl.when(k > 0)
        def _():
            out_desc(c0 - 2, ov_a, semo_a).wait()

        compute(rs_a, rd_a, rw_a, ov_a)
        out_desc(c0, ov_a, semo_a).start()
        fire(row_descs(c0 + 2, rs_a, rd_a, rw_a, semr_a))
        wait(row_descs(c0 + 1, rs_b, rd_b, rw_b, semr_b))

        @pl.when(k > 0)
        def _():
            out_desc(c0 - 1, ov_b, semo_b).wait()

        compute(rs_b, rd_b, rw_b, ov_b)
        out_desc(c0 + 1, ov_b, semo_b).start()
        return carry

    n_steps = (n_chunks - 1) // 2
    lax.fori_loop(0, n_steps, step, 0, unroll=False)

    last = n_chunks - 1
    wait(row_descs(last, rs_a, rd_a, rw_a, semr_a))
    out_desc(last - 2, ov_a, semo_a).wait()
    compute(rs_a, rd_a, rw_a, ov_a)
    out_desc(last, ov_a, semo_a).start()
    out_desc(last, ov_a, semo_a).wait()
    out_desc(last - 1, ov_b, semo_b).wait()


@functools.partial(jax.jit, static_argnames=("B",))
def _simple_scores(x2, wcat, src, dst, et, B=80):
    E = src.shape[0]
    assert E % (_NW * B) == 0 and B % _L == 0
    n_chunks = E // (_NW * B)
    assert n_chunks % 2 == 1 and n_chunks >= 3
    mesh = plsc.VectorSubcoreMesh(core_axis_name="c", subcore_axis_name="s")
    body = functools.partial(_sc_body, B, n_chunks)
    return pl.kernel(
        body,
        out_type=jax.ShapeDtypeStruct((E,), jnp.float32),
        mesh=mesh,
        compiler_params=pltpu.CompilerParams(needs_layout_passes=False),
        scratch_types=[
            pltpu.VMEM((n_chunks * B,), jnp.int32),
            pltpu.VMEM((n_chunks * B,), jnp.int32),
            pltpu.VMEM((n_chunks * B,), jnp.int32),
            pltpu.VMEM((B, _C), jnp.uint32),
            pltpu.VMEM((B, _C), jnp.uint32),
            pltpu.VMEM((B, _D), jnp.float32),
            pltpu.VMEM((B, _C), jnp.uint32),
            pltpu.VMEM((B, _C), jnp.uint32),
            pltpu.VMEM((B, _D), jnp.float32),
            pltpu.VMEM((B,), jnp.float32),
            pltpu.VMEM((B,), jnp.float32),
            pltpu.SemaphoreType.DMA,
            pltpu.SemaphoreType.DMA,
            pltpu.SemaphoreType.DMA,
            pltpu.SemaphoreType.DMA,
            pltpu.SemaphoreType.DMA,
        ],
    )(x2, wcat, src, dst, et)


def kernel(x, edge_index, edge_type, weights, weights_inv, B=80):
    n = x.shape[0]
    x2 = _pack_rows(x.reshape(n, _D))
    wcat = jnp.concatenate([weights, weights_inv], axis=1) * jnp.float32(0.5)
    return _simple_scores(x2, wcat, edge_index[0], edge_index[1], edge_type,
                          B=B)


# speedup vs baseline: 1.0959x; 1.0959x over previous
import functools

import jax
import jax.numpy as jnp
import numpy as np
from jax import lax
from jax.experimental import pallas as pl
from jax.experimental.pallas import tpu as pltpu
from jax.experimental.pallas import tpu_sc as plsc

_NC = 2
_NS = 16
_NW = _NC * _NS
_L = 16
_C = 128
_D = 2 * _C
_HW = _C // 2

_HI_MASK = np.uint32(0xFFFF0000)

_GATHER_DNUMS = lax.GatherDimensionNumbers(
    offset_dims=(), collapsed_slice_dims=(0,), start_index_map=(0,))


def _shuffle(v, idx):
    return lax.gather(v, idx[:, None], _GATHER_DNUMS, (1,),
                      mode=lax.GatherScatterMode.PROMISE_IN_BOUNDS)


def _hsum(v, lanes):
    for k in (8, 4, 2, 1):
        v = v + _shuffle(v, lanes ^ k)
    return v


def _rnd(u):
    return u + np.uint32(0x7FFF) + jnp.bitwise_and(
        jnp.right_shift(u, 16), np.uint32(1))


def _pack_rows(a):
    v = lax.bitcast_convert_type(a, jnp.uint32)
    h = jnp.bitwise_or(
        jnp.bitwise_and(_rnd(v[:, _HW:_C]), _HI_MASK),
        jnp.right_shift(_rnd(v[:, 0:_HW]), 16))
    t = jnp.bitwise_or(
        jnp.bitwise_and(_rnd(v[:, _C + _HW:_D]), _HI_MASK),
        jnp.right_shift(_rnd(v[:, _C:_C + _HW]), 16))
    return jnp.concatenate([h, t], axis=1)


def _sc_body(B, n_chunks, x2, wcat, src, dst, et, out,
             ixs, ixd, ixt, rs_a, rd_a, rw_a, rs_b, rd_b, rw_b, ov_a, ov_b,
             semi, semr_a, semr_b, semo_a, semo_b):
    epw = n_chunks * B
    wid = lax.axis_index("s") * _NC + lax.axis_index("c")
    base = wid * epw
    lanes = lax.iota(jnp.int32, _L)

    def idx_descs():
        return (
            pltpu.make_async_copy(src.at[pl.ds(base, epw)], ixs, semi),
            pltpu.make_async_copy(dst.at[pl.ds(base, epw)], ixd, semi),
            pltpu.make_async_copy(et.at[pl.ds(base, epw)], ixt, semi),
        )

    def row_descs(c, rs, rd, rw, sem):
        off = c * B
        return (
            pltpu.make_async_copy(x2.at[ixs.at[pl.ds(off, B)]], rs, sem),
            pltpu.make_async_copy(x2.at[ixd.at[pl.ds(off, B)]], rd, sem),
            pltpu.make_async_copy(wcat.at[ixt.at[pl.ds(off, B)]], rw, sem),
        )

    def out_desc(c, ov, sem):
        return pltpu.make_async_copy(ov, out.at[pl.ds(base + c * B, B)], sem)

    def fire(descs):
        for d in descs:
            d.start()

    def wait(descs):
        for d in descs:
            d.wait()

    def compute(rs, rd, rw, ov):
        def ext(u):
            hi = plsc.bitcast(jnp.bitwise_and(u, _HI_MASK), jnp.float32)
            lo = plsc.bitcast(jnp.left_shift(u, 16), jnp.float32)
            return hi, lo

        def group(g, gcarry):
            gbase = g * _L
            ovec = jnp.zeros((_L,), jnp.float32)
            for el in range(_L):
                e = gbase + el
                acc1 = jnp.zeros((_L,), jnp.float32)
                acc2 = jnp.zeros((_L,), jnp.float32)
                acc3 = jnp.zeros((_L,), jnp.float32)
                acc4 = jnp.zeros((_L,), jnp.float32)
                for j in range(_HW // _L):
                    lo = j * _L
                    hi = _HW + j * _L
                    sa, sb = ext(rs[e, pl.ds(lo, _L)])
                    wa, wb = ext(rw[e, pl.ds(lo, _L)])
                    da, db = ext(rd[e, pl.ds(hi, _L)])
                    acc1 = acc1 + sa * wa * da
                    acc2 = acc2 + sb * wb * db
                    sa, sb = ext(rs[e, pl.ds(hi, _L)])
                    wa, wb = ext(rw[e, pl.ds(hi, _L)])
                    da, db = ext(rd[e, pl.ds(lo, _L)])
                    acc3 = acc3 + da * wa * sa
                    acc4 = acc4 + db * wb * sb
                acc = (acc1 + acc2) + (acc3 + acc4)
                ovec = jnp.where(lanes == el, _hsum(acc, lanes), ovec)
            ov[pl.ds(gbase, _L)] = ovec
            return gcarry

        lax.fori_loop(0, B // _L, group, 0, unroll=False)

    fire(idx_descs())
    wait(idx_descs())
    fire(row_descs(0, rs_a, rd_a, rw_a, semr_a))

    def step(k, carry):
        c0 = 2 * k
        fire(row_descs(c0 + 1, rs_b, rd_b, rw_b, semr_b))
        wait(row_descs(c0, rs_a, rd_a, rw_a, semr_a))

        @pl.when(k > 0)
        def _():
            out_desc(c0 - 2, ov_a, semo_a).wait()

        compute(rs_a, rd_a, rw_a, ov_a)
        out_desc(c0, ov_a, semo_a).start()
        fire(row_descs(c0 + 2, rs_a, rd_a, rw_a, semr_a))
        wait(row_descs(c0 + 1, rs_b, rd_b, rw_b, semr_b))

        @pl.when(k > 0)
        def _():
            out_desc(c0 - 1, ov_b, semo_b).wait()

        compute(rs_b, rd_b, rw_b, ov_b)
        out_desc(c0 + 1, ov_b, semo_b).start()
        return carry

    n_steps = (n_chunks - 1) // 2
    lax.fori_loop(0, n_steps, step, 0, unroll=False)

    last = n_chunks - 1
    wait(row_descs(last, rs_a, rd_a, rw_a, semr_a))
    out_desc(last - 2, ov_a, semo_a).wait()
    compute(rs_a, rd_a, rw_a, ov_a)
    out_desc(last, ov_a, semo_a).start()
    out_desc(last, ov_a, semo_a).wait()
    out_desc(last - 1, ov_b, semo_b).wait()


@functools.partial(jax.jit, static_argnames=("B",))
def _simple_scores(x2, wcat, src, dst, et, B=80):
    E = src.shape[0]
    assert E % (_NW * B) == 0 and B % _L == 0
    n_chunks = E // (_NW * B)
    assert n_chunks % 2 == 1 and n_chunks >= 3
    mesh = plsc.VectorSubcoreMesh(core_axis_name="c", subcore_axis_name="s")
    body = functools.partial(_sc_body, B, n_chunks)
    return pl.kernel(
        body,
        out_type=jax.ShapeDtypeStruct((E,), jnp.float32),
        mesh=mesh,
        compiler_params=pltpu.CompilerParams(needs_layout_passes=False),
        scratch_types=[
            pltpu.VMEM((n_chunks * B,), jnp.int32),
            pltpu.VMEM((n_chunks * B,), jnp.int32),
            pltpu.VMEM((n_chunks * B,), jnp.int32),
            pltpu.VMEM((B, _C), jnp.uint32),
            pltpu.VMEM((B, _C), jnp.uint32),
            pltpu.VMEM((B, _C), jnp.uint32),
            pltpu.VMEM((B, _C), jnp.uint32),
            pltpu.VMEM((B, _C), jnp.uint32),
            pltpu.VMEM((B, _C), jnp.uint32),
            pltpu.VMEM((B,), jnp.float32),
            pltpu.VMEM((B,), jnp.float32),
            pltpu.SemaphoreType.DMA,
            pltpu.SemaphoreType.DMA,
            pltpu.SemaphoreType.DMA,
            pltpu.SemaphoreType.DMA,
            pltpu.SemaphoreType.DMA,
        ],
    )(x2, wcat, src, dst, et)


def kernel(x, edge_index, edge_type, weights, weights_inv, B=80):
    n = x.shape[0]
    x2 = _pack_rows(x.reshape(n, _D))
    wcat = _pack_rows(
        jnp.concatenate([weights, weights_inv], axis=1) * jnp.float32(0.5))
    return _simple_scores(x2, wcat, edge_index[0], edge_index[1], edge_type,
                          B=B)


# speedup vs baseline: 1.1055x; 1.0087x over previous
import functools

import jax
import jax.numpy as jnp
import numpy as np
from jax import lax
from jax.experimental import pallas as pl
from jax.experimental.pallas import tpu as pltpu
from jax.experimental.pallas import tpu_sc as plsc

_NC = 2
_NS = 16
_NW = _NC * _NS
_L = 16
_C = 128
_D = 2 * _C
_HW = _C // 2

_HI_MASK = np.uint32(0xFFFF0000)

_GATHER_DNUMS = lax.GatherDimensionNumbers(
    offset_dims=(), collapsed_slice_dims=(0,), start_index_map=(0,))


def _shuffle(v, idx):
    return lax.gather(v, idx[:, None], _GATHER_DNUMS, (1,),
                      mode=lax.GatherScatterMode.PROMISE_IN_BOUNDS)


def _hsum(v, lanes):
    for k in (8, 4, 2, 1):
        v = v + _shuffle(v, lanes ^ k)
    return v


def _rnd(u):
    return u + np.uint32(0x7FFF) + jnp.bitwise_and(
        jnp.right_shift(u, 16), np.uint32(1))


def _pack_rows(a):
    v = lax.bitcast_convert_type(a, jnp.uint32)
    h = jnp.bitwise_or(
        jnp.bitwise_and(_rnd(v[:, _HW:_C]), _HI_MASK),
        jnp.right_shift(_rnd(v[:, 0:_HW]), 16))
    t = jnp.bitwise_or(
        jnp.bitwise_and(_rnd(v[:, _C + _HW:_D]), _HI_MASK),
        jnp.right_shift(_rnd(v[:, _C:_C + _HW]), 16))
    return jnp.concatenate([h, t], axis=1)


def _sc_body(B, n_chunks, x2, wcat, src, dst, et, out,
             ixs, ixd, ixt, rs_a, rd_a, rw_a, rs_b, rd_b, rw_b, ov_a, ov_b,
             semi, semr_a, semr_b, semo_a, semo_b):
    epw = n_chunks * B
    wid = lax.axis_index("s") * _NC + lax.axis_index("c")
    base = wid * epw
    lanes = lax.iota(jnp.int32, _L)

    def idx_descs():
        return (
            pltpu.make_async_copy(src.at[pl.ds(base, epw)], ixs, semi),
            pltpu.make_async_copy(dst.at[pl.ds(base, epw)], ixd, semi),
            pltpu.make_async_copy(et.at[pl.ds(base, epw)], ixt, semi),
        )

    def row_descs(c, rs, rd, rw, sem):
        off = c * B
        return (
            pltpu.make_async_copy(x2.at[ixs.at[pl.ds(off, B)]], rs, sem),
            pltpu.make_async_copy(x2.at[ixd.at[pl.ds(off, B)]], rd, sem),
            pltpu.make_async_copy(wcat.at[ixt.at[pl.ds(off, B)]], rw, sem),
        )

    def out_desc(c, ov, sem):
        return pltpu.make_async_copy(ov, out.at[pl.ds(base + c * B, B)], sem)

    def fire(descs):
        for d in descs:
            d.start()

    def wait(descs):
        for d in descs:
            d.wait()

    def compute(rs, rd, rw, ov):
        def ext(u):
            hi = plsc.bitcast(jnp.bitwise_and(u, _HI_MASK), jnp.float32)
            lo = plsc.bitcast(jnp.left_shift(u, 16), jnp.float32)
            return hi, lo

        def group(g, gcarry):
            gbase = g * _L
            vs = []
            for el in range(_L):
                e = gbase + el
                acc1 = jnp.zeros((_L,), jnp.float32)
                acc2 = jnp.zeros((_L,), jnp.float32)
                acc3 = jnp.zeros((_L,), jnp.float32)
                acc4 = jnp.zeros((_L,), jnp.float32)
                for j in range(_HW // _L):
                    lo = j * _L
                    hi = _HW + j * _L
                    sa, sb = ext(rs[e, pl.ds(lo, _L)])
                    wa, wb = ext(rw[e, pl.ds(lo, _L)])
                    da, db = ext(rd[e, pl.ds(hi, _L)])
                    acc1 = acc1 + sa * wa * da
                    acc2 = acc2 + sb * wb * db
                    sa, sb = ext(rs[e, pl.ds(hi, _L)])
                    wa, wb = ext(rw[e, pl.ds(hi, _L)])
                    da, db = ext(rd[e, pl.ds(lo, _L)])
                    acc3 = acc3 + da * wa * sa
                    acc4 = acc4 + db * wb * sb
                vs.append((acc1 + acc2) + (acc3 + acc4))
            for k in (8, 4, 2, 1):
                sel = jnp.bitwise_and(lanes, k) == 0
                half = len(vs) // 2
                nv = []
                for i in range(half):
                    a, b = vs[i], vs[i + half]
                    ma = a + _shuffle(a, lanes ^ k)
                    mb = b + _shuffle(b, lanes ^ k)
                    nv.append(jnp.where(sel, ma, _shuffle(mb, lanes ^ k)))
                vs = nv
            ov[pl.ds(gbase, _L)] = vs[0]
            return gcarry

        lax.fori_loop(0, B // _L, group, 0, unroll=False)

    fire(idx_descs())
    wait(idx_descs())
    fire(row_descs(0, rs_a, rd_a, rw_a, semr_a))

    def step(k, carry):
        c0 = 2 * k
        fire(row_descs(c0 + 1, rs_b, rd_b, rw_b, semr_b))
        wait(row_descs(c0, rs_a, rd_a, rw_a, semr_a))

        @pl.when(k > 0)
        def _():
            out_desc(c0 - 2, ov_a, semo_a).wait()

        compute(rs_a, rd_a, rw_a, ov_a)
        out_desc(c0, ov_a, semo_a).start()
        fire(row_descs(c0 + 2, rs_a, rd_a, rw_a, semr_a))
        wait(row_descs(c0 + 1, rs_b, rd_b, rw_b, semr_b))

        @pl.when(k > 0)
        def _():
            out_desc(c0 - 1, ov_b, semo_b).wait()

        compute(rs_b, rd_b, rw_b, ov_b)
        out_desc(c0 + 1, ov_b, semo_b).start()
        return carry

    n_steps = (n_chunks - 1) // 2
    lax.fori_loop(0, n_steps, step, 0, unroll=False)

    last = n_chunks - 1
    wait(row_descs(last, rs_a, rd_a, rw_a, semr_a))
    out_desc(last - 2, ov_a, semo_a).wait()
    compute(rs_a, rd_a, rw_a, ov_a)
    out_desc(last, ov_a, semo_a).start()
    out_desc(last, ov_a, semo_a).wait()
    out_desc(last - 1, ov_b, semo_b).wait()


@functools.partial(jax.jit, static_argnames=("B",))
def _simple_scores(x2, wcat, src, dst, et, B=80):
    E = et.shape[0]
    assert E % (_NW * B) == 0 and B % _L == 0
    n_chunks = E // (_NW * B)
    assert n_chunks % 2 == 1 and n_chunks >= 3
    mesh = plsc.VectorSubcoreMesh(core_axis_name="c", subcore_axis_name="s")
    body = functools.partial(_sc_body, B, n_chunks)
    return pl.kernel(
        body,
        out_type=jax.ShapeDtypeStruct((E,), jnp.float32),
        mesh=mesh,
        compiler_params=pltpu.CompilerParams(needs_layout_passes=False),
        scratch_types=[
            pltpu.VMEM((n_chunks * B,), jnp.int32),
            pltpu.VMEM((n_chunks * B,), jnp.int32),
            pltpu.VMEM((n_chunks * B,), jnp.int32),
            pltpu.VMEM((B, _C), jnp.uint32),
            pltpu.VMEM((B, _C), jnp.uint32),
            pltpu.VMEM((B, _C), jnp.uint32),
            pltpu.VMEM((B, _C), jnp.uint32),
            pltpu.VMEM((B, _C), jnp.uint32),
            pltpu.VMEM((B, _C), jnp.uint32),
            pltpu.VMEM((B,), jnp.float32),
            pltpu.VMEM((B,), jnp.float32),
            pltpu.SemaphoreType.DMA,
            pltpu.SemaphoreType.DMA,
            pltpu.SemaphoreType.DMA,
            pltpu.SemaphoreType.DMA,
            pltpu.SemaphoreType.DMA,
        ],
    )(x2, wcat, src, dst, et)


def kernel(x, edge_index, edge_type, weights, weights_inv, B=80):
    n = x.shape[0]
    x2 = _pack_rows(x.reshape(n, _D))
    wcat = _pack_rows(
        jnp.concatenate([weights, weights_inv], axis=1) * jnp.float32(0.5))
    return _simple_scores(x2, wcat, edge_index[0], edge_index[1], edge_type,
                          B=B)
